# baseline (speedup 1.0000x reference)
import jax
import jax.numpy as jnp
from jax.experimental import pallas as pl
from jax.experimental.pallas import tpu as pltpu

_NB = 256
_NS = 8


def _bin_embed_kernel(xt_ref, low_ref, high_ref, w_ref, out_ref, obuf, sems):
    K = w_ref.shape[0]
    F = xt_ref.shape[0]
    i2 = pl.program_id(1)
    g2 = pl.num_programs(1)
    chunk = pl.program_id(0) * g2 + i2
    slot = jax.lax.rem(i2, _NS)

    ii = jax.lax.broadcasted_iota(jnp.int32, (K, K), 0)
    kk = jax.lax.broadcasted_iota(jnp.int32, (K, K), 1)
    s = 1.0 / (jnp.abs(ii - kk) + 1).astype(jnp.float32)
    wt = jnp.swapaxes(w_ref[...], 0, 1)
    tt = jnp.dot(wt, s, preferred_element_type=jnp.float32)
    ttb = tt.astype(jnp.bfloat16)

    @pl.when(i2 >= _NS)
    def _wait_slot():
        pltpu.make_async_copy(obuf.at[slot], obuf.at[slot], sems.at[slot]).wait()

    low = low_ref[...]
    high = high_ref[...]
    fq = F // 4
    for f in range(F):
        xrow = xt_ref[f:f + 1, :]
        xb = jnp.broadcast_to(xrow, low.shape)
        m = (xb > low) & (xb <= high)
        oh = jnp.where(m, 1.0, 0.0).astype(jnp.bfloat16)
        obuf[slot, f, :, :] = jnp.dot(ttb, oh, preferred_element_type=jnp.float32)
        if (f + 1) % fq == 0:
            pltpu.make_async_copy(
                obuf.at[slot, pl.ds(f + 1 - fq, fq)],
                out_ref.at[pl.ds(f + 1 - fq, fq), :, pl.ds(chunk * _NB, _NB)],
                sems.at[slot],
            ).start()

    @pl.when(i2 == g2 - 1)
    def _drain():
        for s_ in range(_NS):
            pltpu.make_async_copy(obuf.at[s_], obuf.at[s_], sems.at[s_]).wait()


def kernel(x, low, high, weight):
    B, F = x.shape
    K, D = weight.shape
    xt = x.T
    lowc = jnp.broadcast_to(low[:, None], (K, _NB))
    highc = jnp.broadcast_to(high[:, None], (K, _NB))
    g2 = B // _NB // 2

    out_t = pl.pallas_call(
        _bin_embed_kernel,
        grid=(2, g2),
        in_specs=[
            pl.BlockSpec((F, _NB), lambda c, i: (0, c * (B // _NB // 2) + i)),
            pl.BlockSpec((K, _NB), lambda c, i: (0, 0)),
            pl.BlockSpec((K, _NB), lambda c, i: (0, 0)),
            pl.BlockSpec((K, D), lambda c, i: (0, 0)),
        ],
        out_specs=pl.BlockSpec(memory_space=pl.ANY),
        out_shape=jax.ShapeDtypeStruct((F, D, B), jnp.float32),
        scratch_shapes=[
            pltpu.VMEM((_NS, F, D, _NB), jnp.float32),
            pltpu.SemaphoreType.DMA((_NS,)),
        ],
        compiler_params=pltpu.CompilerParams(
            dimension_semantics=("parallel", "arbitrary"),
            vmem_limit_bytes=48 * 1024 * 1024,
        ),
    )(xt, lowc, highc, weight)
    return jnp.transpose(out_t, (2, 0, 1))


# speedup vs baseline: 1.0056x; 1.0056x over previous
import jax
import jax.numpy as jnp
from jax.experimental import pallas as pl
from jax.experimental.pallas import tpu as pltpu

_NB = 256
_NS = 6


def _bin_embed_kernel(xt_ref, low_ref, high_ref, w_ref, out_ref, obuf, sems):
    K = w_ref.shape[0]
    F = xt_ref.shape[0]
    i2 = pl.program_id(1)
    g2 = pl.num_programs(1)
    chunk = pl.program_id(0) * g2 + i2
    slot = jax.lax.rem(i2, _NS)

    ii = jax.lax.broadcasted_iota(jnp.int32, (K, K), 0)
    kk = jax.lax.broadcasted_iota(jnp.int32, (K, K), 1)
    s = 1.0 / (jnp.abs(ii - kk) + 1).astype(jnp.float32)
    wt = jnp.swapaxes(w_ref[...], 0, 1)
    tt = jnp.dot(wt, s, preferred_element_type=jnp.float32)
    ttb = tt.astype(jnp.bfloat16)

    @pl.when(i2 >= _NS)
    def _wait_slot():
        pltpu.make_async_copy(obuf.at[slot], obuf.at[slot], sems.at[slot]).wait()

    low = low_ref[...]
    high = high_ref[...]
    fq = F // 4
    for f in range(F):
        xrow = xt_ref[f:f + 1, :]
        xb = jnp.broadcast_to(xrow, low.shape)
        m = (xb > low) & (xb <= high)
        oh = jnp.where(m, 1.0, 0.0).astype(jnp.bfloat16)
        obuf[slot, f, :, :] = jnp.dot(ttb, oh, preferred_element_type=jnp.float32)
        if (f + 1) % fq == 0:
            pltpu.make_async_copy(
                obuf.at[slot, pl.ds(f + 1 - fq, fq)],
                out_ref.at[pl.ds(f + 1 - fq, fq), :, pl.ds(chunk * _NB, _NB)],
                sems.at[slot],
            ).start()

    @pl.when(i2 == g2 - 1)
    def _drain():
        for s_ in range(_NS):
            pltpu.make_async_copy(obuf.at[s_], obuf.at[s_], sems.at[s_]).wait()


def kernel(x, low, high, weight):
    B, F = x.shape
    K, D = weight.shape
    xt = x.T
    lowc = jnp.broadcast_to(low[:, None], (K, _NB))
    highc = jnp.broadcast_to(high[:, None], (K, _NB))
    g2 = B // _NB // 2

    out_t = pl.pallas_call(
        _bin_embed_kernel,
        grid=(2, g2),
        in_specs=[
            pl.BlockSpec((F, _NB), lambda c, i: (0, c * (B // _NB // 2) + i)),
            pl.BlockSpec((K, _NB), lambda c, i: (0, 0)),
            pl.BlockSpec((K, _NB), lambda c, i: (0, 0)),
            pl.BlockSpec((K, D), lambda c, i: (0, 0)),
        ],
        out_specs=pl.BlockSpec(memory_space=pl.ANY),
        out_shape=jax.ShapeDtypeStruct((F, D, B), jnp.float32),
        scratch_shapes=[
            pltpu.VMEM((_NS, F, D, _NB), jnp.float32),
            pltpu.SemaphoreType.DMA((_NS,)),
        ],
        compiler_params=pltpu.CompilerParams(
            dimension_semantics=("parallel", "arbitrary"),
            vmem_limit_bytes=48 * 1024 * 1024,
        ),
    )(xt, lowc, highc, weight)
    return jnp.transpose(out_t, (2, 0, 1))
